# baseline (speedup 1.0000x reference)
import functools

import jax
import jax.numpy as jnp
from jax import lax
from jax.experimental import pallas as pl
from jax.experimental.pallas import tpu as pltpu
from jax.experimental.pallas import tpu_sc as plsc

N = 10000
E = 320000
F = 128
CW = 16
CHUNK = 128
EPAD = 2560 * CHUNK
CPW = 2560 // 32
HALF = CPW // 2
NACC = 10240
ROWS_PER_TILE = NACC // 16
CNT_PER_TILE = N // 16
OWN = 632
OWN_LAST = N - 15 * OWN

_f32 = jnp.float32


def _finish_h_body(parts_ref, cnts_ref, wenc_ref, benc_ref, h_ref):
    tot = parts_ref[0] + parts_ref[1]
    cnt = (cnts_ref[0] + cnts_ref[1])[:, 0:1]
    xm = tot * (1.0 / jnp.maximum(cnt, 1.0))
    h = jnp.dot(xm, wenc_ref[...], preferred_element_type=_f32)
    h_ref[...] = jnp.maximum(h + benc_ref[...], 0.0)


def _finish_h(parts, cnts, wenc, benc, blk=1000):
    grid = N // blk
    mat = pl.BlockSpec((F, F), lambda i: (0, 0))
    vec = pl.BlockSpec((1, F), lambda i: (0, 0))
    return pl.pallas_call(
        _finish_h_body,
        grid=(grid,),
        in_specs=[
            pl.BlockSpec((2, blk, F), lambda i: (0, i, 0)),
            pl.BlockSpec((2, blk, CW), lambda i: (0, i, 0)),
            mat, vec,
        ],
        out_specs=pl.BlockSpec((blk, F), lambda i: (i, 0)),
        out_shape=jax.ShapeDtypeStruct((N, F), _f32),
    )(parts, cnts, wenc, benc)


def _mlp_body(in_ref, w1_ref, b1_ref, w2_ref, b2_ref, out_ref):
    t = jnp.dot(in_ref[...], w1_ref[...], preferred_element_type=_f32)
    t = jnp.maximum(t + b1_ref[...], 0.0)
    out_ref[...] = (
        jnp.dot(t, w2_ref[...], preferred_element_type=_f32) + b2_ref[...]
    )


def _mlp(inp, w1, b1, w2, b2, blk=1000):
    grid = N // blk
    mat = pl.BlockSpec((F, F), lambda i: (0, 0))
    vec = pl.BlockSpec((1, F), lambda i: (0, 0))
    return pl.pallas_call(
        _mlp_body,
        grid=(grid,),
        in_specs=[
            pl.BlockSpec((blk, F), lambda i: (i, 0)),
            mat, vec, mat, vec,
        ],
        out_specs=pl.BlockSpec((blk, F), lambda i: (i, 0)),
        out_shape=jax.ShapeDtypeStruct((N, F), _f32),
    )(inp, w1, b1, w2, b2)


def _finish_mlp_body(parts_ref, cnts_ref, w1_ref, b1_ref, w2_ref, b2_ref,
                     out_ref):
    tot = parts_ref[0] + parts_ref[1]
    cnt = (cnts_ref[0] + cnts_ref[1])[:, 0:1]
    hb = tot * (1.0 / jnp.maximum(cnt, 1.0))
    t = jnp.dot(hb, w1_ref[...], preferred_element_type=_f32)
    t = jnp.maximum(t + b1_ref[...], 0.0)
    out_ref[...] = (
        jnp.dot(t, w2_ref[...], preferred_element_type=_f32) + b2_ref[...]
    )


def _finish_mlp(parts, cnts, w1, b1, w2, b2, blk=1000):
    grid = N // blk
    mat = pl.BlockSpec((F, F), lambda i: (0, 0))
    vec = pl.BlockSpec((1, F), lambda i: (0, 0))
    return pl.pallas_call(
        _finish_mlp_body,
        grid=(grid,),
        in_specs=[
            pl.BlockSpec((2, blk, F), lambda i: (0, i, 0)),
            pl.BlockSpec((2, blk, CW), lambda i: (0, i, 0)),
            mat, vec, mat, vec,
        ],
        out_specs=pl.BlockSpec((blk, F), lambda i: (i, 0)),
        out_shape=jax.ShapeDtypeStruct((N, F), _f32),
    )(parts, cnts, w1, b1, w2, b2)


def _fill_zbuf(zbuf, width):
    z = jnp.zeros((16,), _f32)
    for i in range(16):
        for k in range(width // 16):
            zbuf[i, pl.ds(k * 16, 16)] = z


def _zero_rows(zbuf, acc, row0, nrows16, tail):
    def zero_body(i, carry):
        pltpu.sync_copy(zbuf, acc.at[pl.ds(row0 + i * 16, 16)])
        return carry

    lax.fori_loop(0, nrows16, zero_body, 0)
    if tail:
        pltpu.sync_copy(
            zbuf.at[pl.ds(0, tail)],
            acc.at[pl.ds(row0 + nrows16 * 16, tail)],
        )


def _agg_body(table, src2d, dst2d, out, src_v, dst_v, rows_a, rows_b, zbuf,
              acc, sem_a, sem_b):
    c = lax.axis_index("c")
    s = lax.axis_index("s")
    wid = c * 16 + s
    _fill_zbuf(zbuf, F)
    _zero_rows(zbuf, acc, s * ROWS_PER_TILE, ROWS_PER_TILE // 16, 0)
    plsc.subcore_barrier()

    for half in range(2):
        base = wid * CPW + half * HALF
        pltpu.sync_copy(src2d.at[pl.ds(base, HALF)], src_v)
        pltpu.sync_copy(dst2d.at[pl.ds(base, HALF)], dst_v)
        pltpu.async_copy(table.at[src_v.at[0]], rows_a, sem_a)

        def chunk_body(j, carry):
            jj = 2 * j
            pltpu.async_copy(table.at[src_v.at[jj + 1]], rows_b, sem_b)
            pltpu.make_async_copy(table.at[src_v.at[jj]], rows_a, sem_a).wait()
            pltpu.sync_copy(rows_a, acc.at[dst_v.at[jj]], add=True)
            nxt = jnp.minimum(jj + 2, HALF - 1)
            pltpu.async_copy(table.at[src_v.at[nxt]], rows_a, sem_a)
            pltpu.make_async_copy(table.at[src_v.at[jj]], rows_b, sem_b).wait()
            pltpu.sync_copy(rows_b, acc.at[dst_v.at[jj + 1]], add=True)
            return carry

        lax.fori_loop(0, HALF // 2, chunk_body, 0)
        pltpu.make_async_copy(table.at[src_v.at[0]], rows_a, sem_a).wait()
    plsc.subcore_barrier()

    @pl.when(s < 15)
    def _flush_main():
        pltpu.sync_copy(
            acc.at[pl.ds(s * OWN, OWN)],
            out.at[c, pl.ds(s * OWN, OWN)],
        )

    @pl.when(s == 15)
    def _flush_last():
        pltpu.sync_copy(
            acc.at[pl.ds(15 * OWN, OWN_LAST)],
            out.at[c, pl.ds(15 * OWN, OWN_LAST)],
        )


def _aggregate(table, src2d, dst2d):
    mesh = plsc.VectorSubcoreMesh(core_axis_name="c", subcore_axis_name="s")
    kern = functools.partial(
        pl.kernel,
        mesh=mesh,
        out_type=jax.ShapeDtypeStruct((2, N, F), _f32),
        scratch_types=[
            pltpu.VMEM((HALF, CHUNK), jnp.int32),
            pltpu.VMEM((HALF, CHUNK), jnp.int32),
            pltpu.VMEM((CHUNK, F), _f32),
            pltpu.VMEM((CHUNK, F), _f32),
            pltpu.VMEM((16, F), _f32),
            pltpu.VMEM_SHARED((NACC, F), _f32),
            pltpu.SemaphoreType.DMA,
            pltpu.SemaphoreType.DMA,
        ],
    )(_agg_body)
    return kern(table, src2d, dst2d)


def _count_body(dst2d, out_cnt, dst_v, ones_v, zbuf, cacc):
    c = lax.axis_index("c")
    s = lax.axis_index("s")
    wid = c * 16 + s
    _fill_zbuf(zbuf, CW)
    _zero_rows(zbuf, cacc, s * ROWS_PER_TILE, ROWS_PER_TILE // 16, 0)
    onehot = jnp.where(lax.iota(jnp.int32, 16) == 0, 1.0, 0.0).astype(_f32)
    for i in range(CHUNK):
        ones_v[i, pl.ds(0, CW)] = onehot
    plsc.subcore_barrier()

    pltpu.sync_copy(dst2d.at[pl.ds(wid * CPW, CPW)], dst_v)

    def chunk_body(j, carry):
        pltpu.sync_copy(ones_v, cacc.at[dst_v.at[j]], add=True)
        return carry

    lax.fori_loop(0, CPW, chunk_body, 0)
    plsc.subcore_barrier()

    pltpu.sync_copy(
        cacc.at[pl.ds(s * CNT_PER_TILE, CNT_PER_TILE)],
        out_cnt.at[c, pl.ds(s * CNT_PER_TILE, CNT_PER_TILE)],
    )


def _count(dst2d):
    mesh = plsc.VectorSubcoreMesh(core_axis_name="c", subcore_axis_name="s")
    kern = functools.partial(
        pl.kernel,
        mesh=mesh,
        out_type=jax.ShapeDtypeStruct((2, N, CW), _f32),
        scratch_types=[
            pltpu.VMEM((CPW, CHUNK), jnp.int32),
            pltpu.VMEM((CHUNK, CW), _f32),
            pltpu.VMEM((16, CW), _f32),
            pltpu.VMEM_SHARED((NACC, CW), _f32),
        ],
        compiler_params=pltpu.CompilerParams(use_tc_tiling_on_sc=False),
    )(_count_body)
    return kern(dst2d)


def kernel(x, edge_index, W_enc, b_enc, Wx1, bx1, Wx2, bx2, Wh1, bh1, Wh2, bh2):
    pad = EPAD - E
    pad_ids = lax.iota(jnp.int32, pad)
    srcp = jnp.concatenate([edge_index[0], pad_ids % N]).reshape(-1, CHUNK)
    dstp = jnp.concatenate(
        [edge_index[1], N + pad_ids % (NACC - N)]).reshape(-1, CHUNK)

    cnts = _count(dstp)
    parts1 = _aggregate(x, srcp, dstp)
    h_full = _finish_h(parts1, cnts, W_enc, b_enc.reshape(1, F))
    parts2 = _aggregate(h_full, srcp, dstp)
    x_hat = _mlp(h_full, Wx1, bx1.reshape(1, F), Wx2, bx2.reshape(1, F))
    m_hat = _finish_mlp(
        parts2, cnts, Wh1, bh1.reshape(1, F), Wh2, bh2.reshape(1, F),
    )

    return (h_full, x_hat, m_hat)
